# baseline (speedup 1.0000x reference)
import jax
import jax.numpy as jnp
from jax.experimental import pallas as pl
from jax.experimental.pallas import tpu as pltpu

_TEMP = 0.07
_GROUPS = ((0, 2), (2, 17), (17, 53), (53, 89), (89, 116))
_NPROTO = 116
_PPAD = 128
_BB = 1024
_KB = 512


def _body(f_ref, w_ref, b_ref, p1_ref, p2_ref, p3_ref, p4_ref, p5_ref,
          logits_ref, ids_ref, pn_ref, acc_ref):
    sem = w_ref.shape[1]
    k = pl.program_id(1)
    nk = pl.num_programs(1)

    @pl.when((pl.program_id(0) == 0) & (k == 0))
    def _prep_protos():
        p = jnp.concatenate(
            [p1_ref[...], p2_ref[...], p3_ref[...], p4_ref[...], p5_ref[...],
             jnp.zeros((_PPAD - _NPROTO, sem), jnp.float32)], axis=0)
        pnorm = jnp.sqrt(jnp.sum(p * p, axis=-1, keepdims=True))
        pn_ref[...] = p / jnp.maximum(pnorm, 1e-12)

    part = jnp.dot(f_ref[...], w_ref[pl.ds(k * _KB, _KB), :],
                   preferred_element_type=jnp.float32)

    @pl.when(k == 0)
    def _init_acc():
        acc_ref[...] = part

    @pl.when(k > 0)
    def _add_acc():
        acc_ref[...] += part

    @pl.when(k == nk - 1)
    def _epilogue():
        z = acc_ref[...] + b_ref[...].reshape(1, sem)
        znorm = jnp.sqrt(jnp.sum(z * z, axis=-1, keepdims=True))
        zn = z / jnp.maximum(znorm, 1e-12)
        logits = jax.lax.dot_general(
            zn, pn_ref[...], (((1,), (1,)), ((), ())),
            preferred_element_type=jnp.float32,
        ) / _TEMP
        logits_ref[...] = logits[:, :_NPROTO]
        colf = jax.lax.broadcasted_iota(
            jnp.int32, logits.shape, 1).astype(jnp.float32)
        parts = []
        for s, e in _GROUPS:
            mask = (colf >= s) & (colf < e)
            masked = jnp.where(mask, logits, -jnp.inf)
            m = jnp.max(masked, axis=-1, keepdims=True)
            cand = jnp.where(masked == m, colf, float(_PPAD))
            parts.append(jnp.min(cand, axis=-1, keepdims=True) - s)
        ids_ref[...] = jnp.concatenate(parts, axis=1).astype(jnp.int32)


def kernel(features, proj_w, proj_b, proto_gender, proto_hair, proto_top,
           proto_pants, proto_shoes):
    batch, feat = features.shape
    sem = proj_w.shape[1]
    protos = (proto_gender, proto_hair, proto_top, proto_pants, proto_shoes)

    grid = (batch // _BB, feat // _KB)
    all_logits, ids = pl.pallas_call(
        _body,
        grid=grid,
        in_specs=[
            pl.BlockSpec((_BB, _KB), lambda i, k: (i, k)),
            pl.BlockSpec((feat, sem), lambda i, k: (0, 0)),
            pl.BlockSpec((sem,), lambda i, k: (0,)),
        ] + [
            pl.BlockSpec(p.shape, lambda i, k: (0, 0)) for p in protos
        ],
        out_specs=[
            pl.BlockSpec((_BB, _NPROTO), lambda i, k: (i, 0)),
            pl.BlockSpec((_BB, len(_GROUPS)), lambda i, k: (i, 0)),
        ],
        out_shape=[
            jax.ShapeDtypeStruct((batch, _NPROTO), jnp.float32),
            jax.ShapeDtypeStruct((batch, len(_GROUPS)), jnp.int32),
        ],
        scratch_shapes=[
            pltpu.VMEM((_PPAD, sem), jnp.float32),
            pltpu.VMEM((_BB, sem), jnp.float32),
        ],
        compiler_params=pltpu.CompilerParams(
            dimension_semantics=("arbitrary", "arbitrary"),
        ),
    )(features, proj_w, proj_b, *protos)
    return all_logits, ids


# speedup vs baseline: 1.4459x; 1.4459x over previous
import jax
import jax.numpy as jnp
from jax.experimental import pallas as pl
from jax.experimental.pallas import tpu as pltpu

_TEMP = 0.07
_GROUPS = ((0, 2), (2, 17), (17, 53), (53, 89), (89, 116))
_NPROTO = 116
_PPAD = 128
_BB = 1024
_NH = 2


def _body(f_ref, w_ref, b_ref, p1_ref, p2_ref, p3_ref, p4_ref, p5_ref,
          logits_ref, ids_ref, pn_ref):
    sem = w_ref.shape[1]

    @pl.when(pl.program_id(0) == 0)
    def _prep_protos():
        p = jnp.concatenate(
            [p1_ref[...], p2_ref[...], p3_ref[...], p4_ref[...], p5_ref[...],
             jnp.zeros((_PPAD - _NPROTO, sem), jnp.float32)], axis=0)
        pnorm = jnp.sqrt(jnp.sum(p * p, axis=-1, keepdims=True))
        pn_ref[...] = p / jnp.maximum(pnorm, 1e-12)

    f = f_ref[...]
    w = w_ref[...]
    b = b_ref[...].reshape(1, sem)
    pn = pn_ref[...]
    hh = _BB // _NH
    for h in range(_NH):
        z = jnp.dot(f[h * hh:(h + 1) * hh], w,
                    preferred_element_type=jnp.float32)
        z = z + b
        znorm = jnp.sqrt(jnp.sum(z * z, axis=-1, keepdims=True))
        zn = z / jnp.maximum(znorm, 1e-12)
        logits = jax.lax.dot_general(
            zn, pn, (((1,), (1,)), ((), ())),
            preferred_element_type=jnp.float32,
        ) / _TEMP
        logits_ref[h * hh:(h + 1) * hh, :] = logits[:, :_NPROTO]
        colf = jax.lax.broadcasted_iota(
            jnp.int32, logits.shape, 1).astype(jnp.float32)
        parts = []
        for s, e in _GROUPS:
            mask = (colf >= s) & (colf < e)
            masked = jnp.where(mask, logits, -jnp.inf)
            m = jnp.max(masked, axis=-1, keepdims=True)
            cand = jnp.where(masked == m, colf, float(_PPAD))
            parts.append(jnp.min(cand, axis=-1, keepdims=True) - s)
        ids_ref[h * hh:(h + 1) * hh, :] = jnp.concatenate(
            parts, axis=1).astype(jnp.int32)


def kernel(features, proj_w, proj_b, proto_gender, proto_hair, proto_top,
           proto_pants, proto_shoes):
    batch, feat = features.shape
    sem = proj_w.shape[1]
    protos = (proto_gender, proto_hair, proto_top, proto_pants, proto_shoes)

    grid = (batch // _BB,)
    all_logits, ids = pl.pallas_call(
        _body,
        grid=grid,
        in_specs=[
            pl.BlockSpec((_BB, feat), lambda i: (i, 0)),
            pl.BlockSpec((feat, sem), lambda i: (0, 0)),
            pl.BlockSpec((sem,), lambda i: (0,)),
        ] + [
            pl.BlockSpec(p.shape, lambda i: (0, 0)) for p in protos
        ],
        out_specs=[
            pl.BlockSpec((_BB, _NPROTO), lambda i: (i, 0)),
            pl.BlockSpec((_BB, len(_GROUPS)), lambda i: (i, 0)),
        ],
        out_shape=[
            jax.ShapeDtypeStruct((batch, _NPROTO), jnp.float32),
            jax.ShapeDtypeStruct((batch, len(_GROUPS)), jnp.int32),
        ],
        scratch_shapes=[pltpu.VMEM((_PPAD, sem), jnp.float32)],
        compiler_params=pltpu.CompilerParams(
            dimension_semantics=("arbitrary",),
        ),
    )(features, proj_w, proj_b, *protos)
    return all_logits, ids


# speedup vs baseline: 1.4512x; 1.0037x over previous
import jax
import jax.numpy as jnp
from jax.experimental import pallas as pl
from jax.experimental.pallas import tpu as pltpu

_TEMP = 0.07
_GROUPS = ((0, 2), (2, 17), (17, 53), (53, 89), (89, 116))
_NPROTO = 116
_PPAD = 128
_BB = 1024
_NH = 2


def _body(f_ref, w_ref, b_ref, p1_ref, p2_ref, p3_ref, p4_ref, p5_ref,
          logits_ref, ids_ref, pn_ref):
    sem = w_ref.shape[1]

    @pl.when(pl.program_id(0) == 0)
    def _prep_protos():
        p = jnp.concatenate(
            [p1_ref[...], p2_ref[...], p3_ref[...], p4_ref[...], p5_ref[...],
             jnp.zeros((_PPAD - _NPROTO, sem), jnp.float32)], axis=0)
        pnorm = jnp.sqrt(jnp.sum(p * p, axis=-1, keepdims=True))
        pn_ref[...] = p / jnp.maximum(pnorm, 1e-12)

    f = f_ref[...]
    w = w_ref[...]
    b = b_ref[...].reshape(1, sem)
    pn = pn_ref[...]
    hh = _BB // _NH
    for h in range(_NH):
        z = jnp.dot(f[h * hh:(h + 1) * hh], w,
                    preferred_element_type=jnp.float32)
        z = z + b
        znorm = jnp.sqrt(jnp.sum(z * z, axis=-1, keepdims=True))
        zn = z / jnp.maximum(znorm, 1e-12)
        logits = jax.lax.dot_general(
            zn, pn, (((1,), (1,)), ((), ())),
            preferred_element_type=jnp.float32,
        ) / _TEMP
        logits_ref[h * hh:(h + 1) * hh, :] = logits[:, :_NPROTO]
        colf = jax.lax.broadcasted_iota(
            jnp.int32, logits.shape, 1).astype(jnp.float32)
        parts = []
        for s, e in _GROUPS:
            mask = (colf >= s) & (colf < e)
            masked = jnp.where(mask, logits, -jnp.inf)
            m = jnp.max(masked, axis=-1, keepdims=True)
            cand = jnp.where(masked == m, colf, float(_PPAD))
            parts.append(jnp.min(cand, axis=-1, keepdims=True) - s)
        ids_ref[h * hh:(h + 1) * hh, :] = jnp.concatenate(
            parts, axis=1).astype(jnp.int32)


def kernel(features, proj_w, proj_b, proto_gender, proto_hair, proto_top,
           proto_pants, proto_shoes):
    batch, feat = features.shape
    sem = proj_w.shape[1]
    protos = (proto_gender, proto_hair, proto_top, proto_pants, proto_shoes)

    grid = (batch // _BB,)
    all_logits, ids = pl.pallas_call(
        _body,
        grid=grid,
        in_specs=[
            pl.BlockSpec((_BB, feat), lambda i: (i, 0)),
            pl.BlockSpec((feat, sem), lambda i: (0, 0)),
            pl.BlockSpec((sem,), lambda i: (0,)),
        ] + [
            pl.BlockSpec(p.shape, lambda i: (0, 0)) for p in protos
        ],
        out_specs=[
            pl.BlockSpec((_BB, _NPROTO), lambda i: (i, 0)),
            pl.BlockSpec((_BB, len(_GROUPS)), lambda i: (i, 0)),
        ],
        out_shape=[
            jax.ShapeDtypeStruct((batch, _NPROTO), jnp.float32),
            jax.ShapeDtypeStruct((batch, len(_GROUPS)), jnp.int32),
        ],
        scratch_shapes=[pltpu.VMEM((_PPAD, sem), jnp.float32)],
        compiler_params=pltpu.CompilerParams(
            dimension_semantics=("parallel",),
        ),
    )(features, proj_w, proj_b, *protos)
    return all_logits, ids
